# Initial kernel scaffold; baseline (speedup 1.0000x reference)
import jax
import jax.numpy as jnp
from jax.experimental import pallas as pl


def kernel(x, table):
    raise NotImplementedError("write your pallas kernel here")


# speedup vs baseline: 2.1003x; 2.1003x over previous
import functools

import jax
import jax.numpy as jnp
from jax import lax
from jax.experimental import pallas as pl
from jax.experimental.pallas import tpu as pltpu
from jax.experimental.pallas import tpu_sc as plsc

NUM_CORES = 2
NUM_SUBCORES = 16
NUM_WORKERS = NUM_CORES * NUM_SUBCORES
PANEL = 128


def _embed_call(b0, b1, d):
    mesh = plsc.VectorSubcoreMesh(core_axis_name="c", subcore_axis_name="s")
    slabs_per_w = b0 // NUM_WORKERS
    d_slv = d - 2 * PANEL

    @functools.partial(
        pl.kernel,
        mesh=mesh,
        out_type=jax.ShapeDtypeStruct((b0, b1, d), jnp.float32),
        scratch_types=[
            pltpu.VMEM((b1,), jnp.int32),
            pltpu.VMEM((b1, PANEL), jnp.float32),
            pltpu.VMEM((b1, PANEL), jnp.float32),
            pltpu.VMEM((b1, PANEL), jnp.float32),
            pltpu.VMEM((b1, d_slv), jnp.float32),
            pltpu.SemaphoreType.DMA,
        ],
    )
    def emb(x_hbm, p0_hbm, p1_hbm, p2_hbm, out_hbm, idx_v, b0_v, b1_v, b2_v,
            slv_v, sem):
        wid = lax.axis_index("s") * NUM_CORES + lax.axis_index("c")
        base = wid * slabs_per_w

        def body(j, carry):
            b = base + j
            pltpu.sync_copy(x_hbm.at[b], idx_v)
            g0 = pltpu.async_copy(p0_hbm.at[idx_v], b0_v, sem)
            g1 = pltpu.async_copy(p1_hbm.at[idx_v], b1_v, sem)
            g2 = pltpu.async_copy(p2_hbm.at[idx_v], b2_v, sem)
            g0.wait()
            g1.wait()
            g2.wait()
            pltpu.sync_copy(b0_v, out_hbm.at[b, :, pl.ds(0, PANEL)])
            pltpu.sync_copy(b1_v, out_hbm.at[b, :, pl.ds(PANEL, PANEL)])
            for i in range(b1):
                slv_v[i, pl.ds(0, 16)] = b2_v[i, pl.ds(0, 16)]
                slv_v[i, pl.ds(16, 16)] = b2_v[i, pl.ds(16, 16)]
                slv_v[i, pl.ds(d_slv - 16, 16)] = b2_v[i, pl.ds(d_slv - 16, 16)]
            pltpu.sync_copy(slv_v, out_hbm.at[b, :, pl.ds(2 * PANEL, d_slv)])
            return carry

        lax.fori_loop(0, slabs_per_w, body, 0)

    return emb


def kernel(x, table):
    b0, b1 = x.shape
    d = table.shape[1]
    p0 = table[:, :PANEL]
    p1 = table[:, PANEL:2 * PANEL]
    p2 = jnp.pad(table[:, 2 * PANEL:], ((0, 0), (0, 3 * PANEL - d)))
    return _embed_call(b0, b1, d)(x, p0, p1, p2)


# speedup vs baseline: 2.6217x; 1.2482x over previous
import functools

import jax
import jax.numpy as jnp
from jax import lax
from jax.experimental import pallas as pl
from jax.experimental.pallas import tpu as pltpu
from jax.experimental.pallas import tpu_sc as plsc

NUM_CORES = 2
NUM_SUBCORES = 16
NUM_WORKERS = NUM_CORES * NUM_SUBCORES
PANEL = 128
NBUF = 2


def _embed_call(b0, b1, d):
    mesh = plsc.VectorSubcoreMesh(core_axis_name="c", subcore_axis_name="s")
    slabs_per_w = b0 // NUM_WORKERS
    n_groups = slabs_per_w // NBUF
    d_slv = d - 2 * PANEL

    buf_t = pltpu.VMEM((b1, PANEL), jnp.float32)
    scratch = [pltpu.VMEM((slabs_per_w, b1), jnp.int32)]
    for _ in range(NBUF):
        scratch += [buf_t, buf_t, buf_t, pltpu.VMEM((b1, d_slv), jnp.float32),
                    pltpu.SemaphoreType.DMA, pltpu.SemaphoreType.DMA]

    @functools.partial(
        pl.kernel,
        mesh=mesh,
        out_type=jax.ShapeDtypeStruct((b0, b1, d), jnp.float32),
        scratch_types=scratch,
    )
    def emb(x_hbm, p0_hbm, p1_hbm, p2_hbm, out_hbm, idx_all, *scr):
        sets = [scr[i * 6:(i + 1) * 6] for i in range(NBUF)]
        wid = lax.axis_index("s") * NUM_CORES + lax.axis_index("c")
        base = wid * slabs_per_w
        pltpu.sync_copy(x_hbm.at[pl.ds(base, slabs_per_w)], idx_all)

        def writes_of(k, b, issue):
            b0_v, b1_v, b2_v, slv_v, sem_g, sem_w = sets[k]
            mk = pltpu.async_copy if issue else pltpu.make_async_copy
            return [
                mk(b0_v, out_hbm.at[b, :, pl.ds(0, PANEL)], sem_w),
                mk(b1_v, out_hbm.at[b, :, pl.ds(PANEL, PANEL)], sem_w),
                mk(slv_v, out_hbm.at[b, :, pl.ds(2 * PANEL, d_slv)], sem_w),
            ]

        def body(g, carry):
            for k in range(NBUF):
                j = g * NBUF + k
                b = base + j
                b0_v, b1_v, b2_v, slv_v, sem_g, sem_w = sets[k]

                @pl.when(g > 0)
                def _():
                    for w in writes_of(k, b, issue=False):
                        w.wait()

                idx = idx_all.at[j]
                pltpu.async_copy(p0_hbm.at[idx], b0_v, sem_g)
                pltpu.async_copy(p1_hbm.at[idx], b1_v, sem_g)
                pltpu.async_copy(p2_hbm.at[idx], b2_v, sem_g)

            for k in range(NBUF):
                j = g * NBUF + k
                b = base + j
                b0_v, b1_v, b2_v, slv_v, sem_g, sem_w = sets[k]
                pltpu.make_async_copy(p0_hbm.at[idx_all.at[j]], b0_v, sem_g).wait()
                pltpu.make_async_copy(p1_hbm.at[idx_all.at[j]], b1_v, sem_g).wait()
                pltpu.make_async_copy(p2_hbm.at[idx_all.at[j]], b2_v, sem_g).wait()
                for i in range(b1):
                    slv_v[i, pl.ds(0, 16)] = b2_v[i, pl.ds(0, 16)]
                    slv_v[i, pl.ds(16, 16)] = b2_v[i, pl.ds(16, 16)]
                    slv_v[i, pl.ds(d_slv - 16, 16)] = b2_v[i, pl.ds(d_slv - 16, 16)]
                writes_of(k, b, issue=True)
            return carry

        lax.fori_loop(0, n_groups, body, 0)
        for k in range(NBUF):
            b = base + (n_groups - 1) * NBUF + k
            for w in writes_of(k, b, issue=False):
                w.wait()

    return emb


def kernel(x, table):
    b0, b1 = x.shape
    d = table.shape[1]
    p0 = table[:, :PANEL]
    p1 = table[:, PANEL:2 * PANEL]
    p2 = jnp.pad(table[:, 2 * PANEL:], ((0, 0), (0, 3 * PANEL - d)))
    return _embed_call(b0, b1, d)(x, p0, p1, p2)


# speedup vs baseline: 3.3054x; 1.2608x over previous
import functools

import jax
import jax.numpy as jnp
from jax import lax
from jax.experimental import pallas as pl
from jax.experimental.pallas import tpu as pltpu
from jax.experimental.pallas import tpu_sc as plsc

NUM_CORES = 2
NUM_SUBCORES = 16
NUM_WORKERS = NUM_CORES * NUM_SUBCORES
PANEL = 128
NBUF = 2


def _embed_call(b0, b1, d):
    mesh = plsc.VectorSubcoreMesh(core_axis_name="c", subcore_axis_name="s")
    slabs_per_w = b0 // NUM_WORKERS
    n_groups = slabs_per_w // NBUF
    d_slv = d - 2 * PANEL

    buf_t = pltpu.VMEM((b1, PANEL), jnp.float32)
    scratch = [pltpu.VMEM((slabs_per_w, b1), jnp.int32)]
    for _ in range(NBUF):
        scratch += [buf_t, buf_t, buf_t, pltpu.VMEM((b1, d_slv), jnp.float32),
                    pltpu.SemaphoreType.DMA, pltpu.SemaphoreType.DMA]
    scratch += [pltpu.VMEM_SHARED((1000, PANEL), jnp.float32) for _ in range(3)]

    @functools.partial(
        pl.kernel,
        mesh=mesh,
        out_type=jax.ShapeDtypeStruct((b0, b1, d), jnp.float32),
        scratch_types=scratch,
    )
    def emb(x_hbm, p0_hbm, p1_hbm, p2_hbm, out_hbm, idx_all, *scr):
        sets = [scr[i * 6:(i + 1) * 6] for i in range(NBUF)]
        sp0, sp1, sp2 = scr[NBUF * 6:NBUF * 6 + 3]
        sid = lax.axis_index("s")
        wid = sid * NUM_CORES + lax.axis_index("c")
        base = wid * slabs_per_w

        @pl.when(sid == 0)
        def _():
            pltpu.sync_copy(p0_hbm, sp0)
            pltpu.sync_copy(p1_hbm, sp1)
            pltpu.sync_copy(p2_hbm, sp2)

        pltpu.sync_copy(x_hbm.at[pl.ds(base, slabs_per_w)], idx_all)
        plsc.subcore_barrier()

        def writes_of(k, b, issue):
            b0_v, b1_v, b2_v, slv_v, sem_g, sem_w = sets[k]
            mk = pltpu.async_copy if issue else pltpu.make_async_copy
            return [
                mk(b0_v, out_hbm.at[b, :, pl.ds(0, PANEL)], sem_w),
                mk(b1_v, out_hbm.at[b, :, pl.ds(PANEL, PANEL)], sem_w),
                mk(slv_v, out_hbm.at[b, :, pl.ds(2 * PANEL, d_slv)], sem_w),
            ]

        def body(g, carry):
            for k in range(NBUF):
                j = g * NBUF + k
                b = base + j
                b0_v, b1_v, b2_v, slv_v, sem_g, sem_w = sets[k]

                @pl.when(g > 0)
                def _():
                    for w in writes_of(k, b, issue=False):
                        w.wait()

                idx = idx_all.at[j]
                pltpu.async_copy(sp0.at[idx], b0_v, sem_g)
                pltpu.async_copy(sp1.at[idx], b1_v, sem_g)
                pltpu.async_copy(sp2.at[idx], b2_v, sem_g)

            for k in range(NBUF):
                j = g * NBUF + k
                b = base + j
                b0_v, b1_v, b2_v, slv_v, sem_g, sem_w = sets[k]
                pltpu.make_async_copy(sp0.at[idx_all.at[j]], b0_v, sem_g).wait()
                pltpu.make_async_copy(sp1.at[idx_all.at[j]], b1_v, sem_g).wait()
                pltpu.make_async_copy(sp2.at[idx_all.at[j]], b2_v, sem_g).wait()
                for i in range(b1):
                    slv_v[i, pl.ds(0, 16)] = b2_v[i, pl.ds(0, 16)]
                    slv_v[i, pl.ds(16, 16)] = b2_v[i, pl.ds(16, 16)]
                    slv_v[i, pl.ds(d_slv - 16, 16)] = b2_v[i, pl.ds(d_slv - 16, 16)]
                writes_of(k, b, issue=True)
            return carry

        lax.fori_loop(0, n_groups, body, 0)
        for k in range(NBUF):
            b = base + (n_groups - 1) * NBUF + k
            for w in writes_of(k, b, issue=False):
                w.wait()

    return emb


def kernel(x, table):
    b0, b1 = x.shape
    d = table.shape[1]
    p0 = table[:, :PANEL]
    p1 = table[:, PANEL:2 * PANEL]
    p2 = jnp.pad(table[:, 2 * PANEL:], ((0, 0), (0, 3 * PANEL - d)))
    return _embed_call(b0, b1, d)(x, p0, p1, p2)
